# baseline (device time: 74086 ns/iter reference)
import jax
import jax.numpy as jnp
from jax import lax
from jax.experimental import pallas as pl
from jax.experimental.pallas import tpu as pltpu

import os

SKIP_COMM = os.environ.get("SKIP_COMM") == "1"
PROF = os.environ.get("PROF_SCOPES") == "1"

import contextlib

def _scope(name):
    return jax.named_scope(name) if PROF else contextlib.nullcontext()

N_DEV = 16
N_ROUNDS = 4

MASK_XOR = {"x": 1, "y": 3, "z0": 4, "z1": 8}

STREAMS = (
    (448, ("x", "y", "z0", "z1")),
    (448, ("y", "x", "z1", "z0")),
    (384, ("z0", "z1", "x", "y")),
    (256, ("z1", "z0", "y", "x")),
)
N_STREAMS = len(STREAMS)

RS_WAIT_ORDER = ((2, 0, 1, 3), (3, 0, 1, 2), (3, 2, 0, 1), (3, 2, 1, 0))
AG_WAIT_ORDER = ((3, 2, 1, 0), (3, 2, 0, 1), (3, 0, 1, 2), (2, 0, 1, 3))

STAGE_OFF = (0, 8, 12, 14)


def _bit(mask, p):
    if mask == "x":
        return (p & 1) ^ ((p >> 1) & 1)
    if mask == "y":
        return (p >> 1) & 1
    if mask == "z0":
        return (p >> 2) & 1
    return (p >> 3) & 1


def _chunk_of(p, order):
    c = 0
    for j, m in enumerate(order):
        c |= _bit(m, p) << (3 - j)
    return c


def kernel(A, B):
    M, K = A.shape
    _, N = B.shape
    C = M // N_DEV
    col_off = [0]
    for w, _ in STREAMS:
        col_off.append(col_off[-1] + w)
    assert col_off[-1] == N

    def body(a_ref, b_ref, out_ref, *rest):
        works = rest[:N_STREAMS]
        stages = rest[N_STREAMS:2 * N_STREAMS]
        rs_send, rs_recv, ag_send, ag_recv = rest[2 * N_STREAMS:]

        me = lax.axis_index("i")

        sc_barrier = _scope("barrier"); sc_barrier.__enter__()
        barrier_sem = pltpu.get_barrier_semaphore()
        for m in ("x", "y", "z0", "z1"):
            pl.semaphore_signal(
                barrier_sem, inc=1,
                device_id=(me ^ MASK_XOR[m],),
                device_id_type=pl.DeviceIdType.MESH,
            )
        pl.semaphore_wait(barrier_sem, N_ROUNDS)
        sc_barrier.__exit__(None, None, None)

        offs = [None] * N_STREAMS
        rs_rdmas = [None] * N_STREAMS
        ag_rdmas = [None] * N_STREAMS
        ag_sizes = [1] * N_STREAMS

        def start_rs(s, j):
            w, order = STREAMS[s]
            h = 8 >> j
            b = _bit(order[j], me)
            send_off = offs[s] + h * (1 - b)
            offs[s] = offs[s] + h * b
            rdma = pltpu.make_async_remote_copy(
                src_ref=works[s].at[pl.ds(send_off * C, h * C)],
                dst_ref=stages[s].at[pl.ds(STAGE_OFF[j] * C, h * C)],
                send_sem=rs_send.at[s * N_ROUNDS + j],
                recv_sem=rs_recv.at[s * N_ROUNDS + j],
                device_id=(me ^ MASK_XOR[order[j]],),
                device_id_type=pl.DeviceIdType.MESH,
            )
            if not SKIP_COMM:
                rdma.start()
            rs_rdmas[s] = rdma

        def start_ag(s, t):
            _, order = STREAMS[s]
            wgt = 1 << t
            rdma = pltpu.make_async_remote_copy(
                src_ref=works[s].at[pl.ds(offs[s] * C, wgt * C)],
                dst_ref=works[s].at[pl.ds(offs[s] * C, wgt * C)],
                send_sem=ag_send.at[s * N_ROUNDS + t],
                recv_sem=ag_recv.at[s * N_ROUNDS + t],
                device_id=(me ^ MASK_XOR[order[N_ROUNDS - 1 - t]],),
                device_id_type=pl.DeviceIdType.MESH,
            )
            if not SKIP_COMM:
                rdma.start()
            ag_rdmas[s] = rdma

        for s, (w, order) in enumerate(STREAMS):
            inv = [0] * N_DEV
            for c in range(N_DEV):
                inv[_chunk_of(c, order)] = c
            a_perm = jnp.concatenate(
                [a_ref[pl.ds(inv[j] * C, C), :] for j in range(N_DEV)]
            ).astype(jnp.bfloat16)
            works[s][...] = jnp.dot(
                a_perm,
                b_ref[:, pl.ds(col_off[s], w)].astype(jnp.bfloat16),
                preferred_element_type=jnp.float32,
            ).astype(jnp.bfloat16)
            offs[s] = jnp.int32(0)
            start_rs(s, 0)

        for j in range(N_ROUNDS):
            h = 8 >> j
            for s in RS_WAIT_ORDER[j]:
                if not SKIP_COMM:
                    with _scope(f"rswait#s={s},j={j}"):
                        rs_rdmas[s].wait()
                keep = works[s][pl.ds(offs[s] * C, h * C)]
                works[s][pl.ds(offs[s] * C, h * C)] = (
                    keep + stages[s][pl.ds(STAGE_OFF[j] * C, h * C)]
                )
                if j + 1 < N_ROUNDS:
                    start_rs(s, j + 1)
                else:
                    z = works[s][pl.ds(offs[s] * C, C)].astype(jnp.float32)
                    works[s][pl.ds(offs[s] * C, C)] = (
                        z / (1.0 + jnp.exp(-z))
                    ).astype(jnp.bfloat16)
                    start_ag(s, 0)

        for t in range(N_ROUNDS):
            if t == N_ROUNDS - 1:
                for s, (w, order) in enumerate(STREAMS):
                    held_base = 8 * _bit(order[0], me)
                    for c in range(N_DEV):
                        cc = _chunk_of(c, order)
                        @pl.when(((cc >= held_base) & (cc < held_base + 8)))
                        def _(s=s, c=c, cc=cc, w=w):
                            out_ref[pl.ds(c * C, C), pl.ds(col_off[s], w)] = (
                                works[s][pl.ds(cc * C, C)]
                            )
            for s in AG_WAIT_ORDER[t]:
                _, order = STREAMS[s]
                if not SKIP_COMM:
                    with _scope(f"agwait#s={s},t={t}"):
                        ag_rdmas[s].wait()
                offs[s] = offs[s] - _bit(order[N_ROUNDS - 1 - t], me) * (1 << t)
                if t + 1 < N_ROUNDS:
                    start_ag(s, t + 1)
                else:
                    w, _ = STREAMS[s]
                    recv_base = 8 * (1 - _bit(order[0], me))
                    for c in range(N_DEV):
                        cc = _chunk_of(c, order)
                        @pl.when(((cc >= recv_base) & (cc < recv_base + 8)))
                        def _(s=s, c=c, cc=cc, w=w):
                            out_ref[pl.ds(c * C, C), pl.ds(col_off[s], w)] = (
                                works[s][pl.ds(cc * C, C)]
                            )

    return pl.pallas_call(
        body,
        out_shape=jax.ShapeDtypeStruct((M, N), jnp.bfloat16),
        in_specs=[
            pl.BlockSpec(memory_space=pltpu.VMEM),
            pl.BlockSpec(memory_space=pltpu.VMEM),
        ],
        out_specs=pl.BlockSpec(memory_space=pltpu.VMEM),
        scratch_shapes=(
            [pltpu.VMEM((M, w), jnp.bfloat16) for w, _ in STREAMS]
            + [pltpu.VMEM((15 * C, w), jnp.bfloat16) for w, _ in STREAMS]
            + [pltpu.SemaphoreType.DMA((N_STREAMS * N_ROUNDS,))] * 4
        ),
        compiler_params=pltpu.CompilerParams(collective_id=0),
    )(A, B)


# device time: 73160 ns/iter; 1.0127x vs baseline; 1.0127x over previous
import jax
import jax.numpy as jnp
from jax import lax
from jax.experimental import pallas as pl
from jax.experimental.pallas import tpu as pltpu

import os

SKIP_COMM = os.environ.get("SKIP_COMM") == "1"
PROF = os.environ.get("PROF_SCOPES") == "1"

import contextlib

def _scope(name):
    return jax.named_scope(name) if PROF else contextlib.nullcontext()

N_DEV = 16
N_ROUNDS = 4

MASK_XOR = {"x": 1, "y": 3, "z0": 4, "z1": 8}

STREAMS = (
    (448, ("x", "y", "z0", "z1")),
    (448, ("y", "x", "z1", "z0")),
    (384, ("z0", "z1", "x", "y")),
    (256, ("z1", "z0", "y", "x")),
)
N_STREAMS = len(STREAMS)

RS_WAIT_ORDER = ((2, 0, 1, 3), (3, 0, 1, 2), (3, 2, 0, 1), (3, 2, 1, 0))
AG_WAIT_ORDER = ((3, 2, 1, 0), (3, 2, 0, 1), (3, 0, 1, 2), (2, 0, 1, 3))

STAGE_OFF = (0, 8, 12, 14)


def _bit(mask, p):
    if mask == "x":
        return (p & 1) ^ ((p >> 1) & 1)
    if mask == "y":
        return (p >> 1) & 1
    if mask == "z0":
        return (p >> 2) & 1
    return (p >> 3) & 1


def _chunk_of(p, order):
    c = 0
    for j, m in enumerate(order):
        c |= _bit(m, p) << (3 - j)
    return c


def kernel(A, B):
    M, K = A.shape
    _, N = B.shape
    C = M // N_DEV
    col_off = [0]
    for w, _ in STREAMS:
        col_off.append(col_off[-1] + w)
    assert col_off[-1] == N

    def body(a_ref, b_ref, out_ref, *rest):
        works = rest[:N_STREAMS]
        stages = rest[N_STREAMS:2 * N_STREAMS]
        rs_send, rs_recv, ag_send, ag_recv = rest[2 * N_STREAMS:]

        me = lax.axis_index("i")

        sc_barrier = _scope("barrier"); sc_barrier.__enter__()
        barrier_sem = pltpu.get_barrier_semaphore()
        for m in ("x", "y", "z0", "z1"):
            pl.semaphore_signal(
                barrier_sem, inc=1,
                device_id=(me ^ MASK_XOR[m],),
                device_id_type=pl.DeviceIdType.MESH,
            )
        pl.semaphore_wait(barrier_sem, N_ROUNDS)
        sc_barrier.__exit__(None, None, None)

        offs = [None] * N_STREAMS
        rs_rdmas = [None] * N_STREAMS
        ag_rdmas = [None] * N_STREAMS
        ag_sizes = [1] * N_STREAMS

        def start_rs(s, j):
            w, order = STREAMS[s]
            if j == N_ROUNDS - 1:
                h = 2
                send_off = offs[s]
            else:
                h = 8 >> j
                b = _bit(order[j], me)
                send_off = offs[s] + h * (1 - b)
                offs[s] = offs[s] + h * b
            rdma = pltpu.make_async_remote_copy(
                src_ref=works[s].at[pl.ds(send_off * C, h * C)],
                dst_ref=stages[s].at[pl.ds(STAGE_OFF[j] * C, h * C)],
                send_sem=rs_send.at[s * N_ROUNDS + j],
                recv_sem=rs_recv.at[s * N_ROUNDS + j],
                device_id=(me ^ MASK_XOR[order[j]],),
                device_id_type=pl.DeviceIdType.MESH,
            )
            if not SKIP_COMM:
                rdma.start()
            rs_rdmas[s] = rdma

        def start_ag(s, t):
            _, order = STREAMS[s]
            wgt = 1 << t
            rdma = pltpu.make_async_remote_copy(
                src_ref=works[s].at[pl.ds(offs[s] * C, wgt * C)],
                dst_ref=works[s].at[pl.ds(offs[s] * C, wgt * C)],
                send_sem=ag_send.at[s * N_ROUNDS + t],
                recv_sem=ag_recv.at[s * N_ROUNDS + t],
                device_id=(me ^ MASK_XOR[order[N_ROUNDS - 1 - t]],),
                device_id_type=pl.DeviceIdType.MESH,
            )
            if not SKIP_COMM:
                rdma.start()
            ag_rdmas[s] = rdma

        for s, (w, order) in enumerate(STREAMS):
            inv = [0] * N_DEV
            for c in range(N_DEV):
                inv[_chunk_of(c, order)] = c
            a_perm = jnp.concatenate(
                [a_ref[pl.ds(inv[j] * C, C), :] for j in range(N_DEV)]
            )
            works[s][...] = jnp.dot(
                a_perm,
                b_ref[:, pl.ds(col_off[s], w)],
                preferred_element_type=jnp.float32,
            ).astype(jnp.bfloat16)
            offs[s] = jnp.int32(0)
            start_rs(s, 0)

        for j in range(N_ROUNDS):
            h = 2 if j == N_ROUNDS - 1 else 8 >> j
            for s in RS_WAIT_ORDER[j]:
                if not SKIP_COMM:
                    with _scope(f"rswait#s={s},j={j}"):
                        rs_rdmas[s].wait()
                keep = works[s][pl.ds(offs[s] * C, h * C)]
                works[s][pl.ds(offs[s] * C, h * C)] = (
                    keep + stages[s][pl.ds(STAGE_OFF[j] * C, h * C)]
                )
                if j + 1 < N_ROUNDS:
                    start_rs(s, j + 1)
                else:
                    z = works[s][pl.ds(offs[s] * C, 2 * C)].astype(jnp.float32)
                    works[s][pl.ds(offs[s] * C, 2 * C)] = (
                        z / (1.0 + jnp.exp(-z))
                    ).astype(jnp.bfloat16)
                    start_ag(s, 1)

        for t in range(1, N_ROUNDS):
            if t == N_ROUNDS - 1:
                for s, (w, order) in enumerate(STREAMS):
                    held_base = 8 * _bit(order[0], me)
                    for c in range(N_DEV):
                        cc = _chunk_of(c, order)
                        @pl.when(((cc >= held_base) & (cc < held_base + 8)))
                        def _(s=s, c=c, cc=cc, w=w):
                            out_ref[pl.ds(c * C, C), pl.ds(col_off[s], w)] = (
                                works[s][pl.ds(cc * C, C)]
                            )
            for s in AG_WAIT_ORDER[t]:
                _, order = STREAMS[s]
                if not SKIP_COMM:
                    with _scope(f"agwait#s={s},t={t}"):
                        ag_rdmas[s].wait()
                offs[s] = offs[s] - _bit(order[N_ROUNDS - 1 - t], me) * (1 << t)
                if t + 1 < N_ROUNDS:
                    start_ag(s, t + 1)
                else:
                    w, _ = STREAMS[s]
                    recv_base = 8 * (1 - _bit(order[0], me))
                    for c in range(N_DEV):
                        cc = _chunk_of(c, order)
                        @pl.when(((cc >= recv_base) & (cc < recv_base + 8)))
                        def _(s=s, c=c, cc=cc, w=w):
                            out_ref[pl.ds(c * C, C), pl.ds(col_off[s], w)] = (
                                works[s][pl.ds(cc * C, C)]
                            )

    return pl.pallas_call(
        body,
        out_shape=jax.ShapeDtypeStruct((M, N), jnp.bfloat16),
        in_specs=[
            pl.BlockSpec(memory_space=pltpu.VMEM),
            pl.BlockSpec(memory_space=pltpu.VMEM),
        ],
        out_specs=pl.BlockSpec(memory_space=pltpu.VMEM),
        scratch_shapes=(
            [pltpu.VMEM((M, w), jnp.bfloat16) for w, _ in STREAMS]
            + [pltpu.VMEM((16 * C, w), jnp.bfloat16) for w, _ in STREAMS]
            + [pltpu.SemaphoreType.DMA((N_STREAMS * N_ROUNDS,))] * 4
        ),
        compiler_params=pltpu.CompilerParams(collective_id=0),
    )(A.astype(jnp.bfloat16), B.astype(jnp.bfloat16))
